# Initial kernel scaffold; baseline (speedup 1.0000x reference)
import jax
import jax.numpy as jnp
from jax.experimental import pallas as pl


def kernel(x, edge_index, W1l, b1, W1r, W2l, b2, W2r):
    raise NotImplementedError("write your pallas kernel here")


# speedup vs baseline: 13.3511x; 13.3511x over previous
import functools

import jax
import jax.numpy as jnp
from jax import lax
from jax.experimental import pallas as pl
from jax.experimental.pallas import tpu as pltpu
from jax.experimental.pallas import tpu_sc as plsc

N_NODES = 10000
D_FEAT = 128
HIDDEN = 16
D_OUT = 128

NC = 2
NS = 16
NW = NC * NS
CH = 128
RPS = 632
NROWS = RPS * NS


def _sc_segsum_body(n_chunks, with_count, table, srcw, dstw, *rest):
    if with_count:
        (agg_out, cnt_out, acc_sh, cnt_sh, src_v, dst_v, rows_v, ones_v,
         bounce_v, sem) = rest
    else:
        (agg_out, acc_sh, src_v, dst_v, rows_v, bounce_v, sem) = rest
    c = lax.axis_index("c")
    s = lax.axis_index("s")
    wid = c * NS + s

    z16 = jnp.zeros((HIDDEN,), jnp.float32)

    def zrow(i, carry):
        bounce_v[i] = z16
        return carry

    lax.fori_loop(0, RPS, zrow, 0)
    pltpu.sync_copy(bounce_v, acc_sh.at[pl.ds(s * RPS, RPS)])
    if with_count:
        pltpu.sync_copy(bounce_v, cnt_sh.at[pl.ds(s * RPS, RPS)])
        o16 = jnp.ones((HIDDEN,), jnp.float32)

        def orow(i, carry):
            ones_v[i] = o16
            return carry

        lax.fori_loop(0, CH, orow, 0)

    pltpu.sync_copy(srcw.at[wid], src_v)
    pltpu.sync_copy(dstw.at[wid], dst_v)
    plsc.subcore_barrier()

    def chunk(j, carry):
        pltpu.async_copy(table.at[src_v.at[j]], rows_v, sem).wait()
        pltpu.sync_copy(rows_v, acc_sh.at[dst_v.at[j]], add=True)
        if with_count:
            pltpu.sync_copy(ones_v, cnt_sh.at[dst_v.at[j]], add=True)
        return carry

    lax.fori_loop(0, n_chunks, chunk, 0)
    plsc.subcore_barrier()

    pltpu.sync_copy(acc_sh.at[pl.ds(s * RPS, RPS)], bounce_v)
    pltpu.sync_copy(bounce_v, agg_out.at[c].at[pl.ds(s * RPS, RPS)])
    if with_count:
        pltpu.sync_copy(cnt_sh.at[pl.ds(s * RPS, RPS)], bounce_v)
        pltpu.sync_copy(bounce_v, cnt_out.at[c].at[pl.ds(s * RPS, RPS)])


@functools.lru_cache(maxsize=None)
def _make_sc_pass(n_chunks, with_count):
    mesh = plsc.VectorSubcoreMesh(core_axis_name="c", subcore_axis_name="s",
                                  num_cores=NC, num_subcores=NS)
    acc_t = jax.ShapeDtypeStruct((NC, NROWS, HIDDEN), jnp.float32)
    out_type = [acc_t, acc_t] if with_count else [acc_t]
    scratch = [pltpu.VMEM_SHARED((NROWS, HIDDEN), jnp.float32)]
    if with_count:
        scratch.append(pltpu.VMEM_SHARED((NROWS, HIDDEN), jnp.float32))
    scratch += [
        pltpu.VMEM((n_chunks, CH), jnp.int32),
        pltpu.VMEM((n_chunks, CH), jnp.int32),
        pltpu.VMEM((CH, HIDDEN), jnp.float32),
    ]
    if with_count:
        scratch.append(pltpu.VMEM((CH, HIDDEN), jnp.float32))
    scratch += [
        pltpu.VMEM((RPS, HIDDEN), jnp.float32),
        pltpu.SemaphoreType.DMA,
    ]
    body = functools.partial(_sc_segsum_body, n_chunks, with_count)
    return pl.kernel(body, out_type=out_type, mesh=mesh,
                     scratch_types=scratch,
                     compiler_params=pltpu.CompilerParams(
                         use_tc_tiling_on_sc=False))


def _proj1_body(x_ref, wl_ref, wr_ref, p_ref, r_ref):
    x = x_ref[...]
    p_ref[...] = jnp.dot(x, wl_ref[...], preferred_element_type=jnp.float32)
    r_ref[...] = jnp.dot(x, wr_ref[...], preferred_element_type=jnp.float32)


def _h_body(agg_ref, cnt_ref, xr_ref, b1_ref, h_ref):
    a = agg_ref[0] + agg_ref[1]
    cw = cnt_ref[0] + cnt_ref[1]
    m = a[:N_NODES] / jnp.maximum(cw[:N_NODES], 1.0)
    h_ref[...] = jnp.maximum(m + xr_ref[...] + b1_ref[...], 0.0)


def _out_body(agg_ref, cnt_ref, h_ref, wl_ref, wr_ref, b2_ref, out_ref):
    a = agg_ref[0] + agg_ref[1]
    cw = cnt_ref[0] + cnt_ref[1]
    m = a[:N_NODES] / jnp.maximum(cw[:N_NODES], 1.0)
    h = h_ref[...]
    out_ref[...] = (jnp.dot(m, wl_ref[...], preferred_element_type=jnp.float32)
                    + jnp.dot(h, wr_ref[...], preferred_element_type=jnp.float32)
                    + b2_ref[...])


_proj1 = pl.pallas_call(
    _proj1_body,
    out_shape=[jax.ShapeDtypeStruct((N_NODES, HIDDEN), jnp.float32),
               jax.ShapeDtypeStruct((N_NODES, HIDDEN), jnp.float32)])

_hstep = pl.pallas_call(
    _h_body,
    out_shape=jax.ShapeDtypeStruct((N_NODES, HIDDEN), jnp.float32))

_outstep = pl.pallas_call(
    _out_body,
    out_shape=jax.ShapeDtypeStruct((N_NODES, D_OUT), jnp.float32))


def kernel(x, edge_index, W1l, b1, W1r, W2l, b2, W2r):
    n_edges = edge_index.shape[1]
    n_chunks = -(-n_edges // (NW * CH))
    e_pad = NW * n_chunks * CH

    src = edge_index[0].astype(jnp.int32)
    dst = edge_index[1].astype(jnp.int32)
    pad = e_pad - n_edges
    if pad:
        src = jnp.concatenate([src, jnp.zeros((pad,), jnp.int32)])
        dst = jnp.concatenate(
            [dst, N_NODES + (jnp.arange(pad, dtype=jnp.int32) % (NROWS - N_NODES))])
    srcw = src.reshape(NW, n_chunks, CH)
    dstw = dst.reshape(NW, n_chunks, CH)

    p1, xr = _proj1(x, W1l.T, W1r.T)
    agg1, cntw = _make_sc_pass(n_chunks, True)(p1, srcw, dstw)
    h = _hstep(agg1, cntw, xr, b1.reshape(1, HIDDEN))
    agg2, = _make_sc_pass(n_chunks, False)(h, srcw, dstw)
    out = _outstep(agg2, cntw, h, W2l.T, W2r.T, b2.reshape(1, D_OUT))
    return out


# speedup vs baseline: 16.2522x; 1.2173x over previous
import functools

import jax
import jax.numpy as jnp
from jax import lax
from jax.experimental import pallas as pl
from jax.experimental.pallas import tpu as pltpu
from jax.experimental.pallas import tpu_sc as plsc

N_NODES = 10000
D_FEAT = 128
HIDDEN = 16
D_OUT = 128

NC = 2
NS = 16
NW = NC * NS
CH = 128
GC = 8
CHG = CH * GC
RPS = 632
NROWS = RPS * NS


def _sc_segsum_body(n_groups, with_count, table, srcw, dstw, *rest):
    if with_count:
        (agg_out, cnt_out, acc_sh, cnt_sh, src_v, dst_v, rows0_v, rows1_v,
         ones_v, bounce_v, gsem0, gsem1) = rest
    else:
        (agg_out, acc_sh, src_v, dst_v, rows0_v, rows1_v, bounce_v,
         gsem0, gsem1) = rest
    c = lax.axis_index("c")
    s = lax.axis_index("s")
    wid = c * NS + s

    z16 = jnp.zeros((HIDDEN,), jnp.float32)

    def zrow(i, carry):
        bounce_v[i] = z16
        return carry

    lax.fori_loop(0, RPS, zrow, 0)
    pltpu.sync_copy(bounce_v, acc_sh.at[pl.ds(s * RPS, RPS)])
    if with_count:
        pltpu.sync_copy(bounce_v, cnt_sh.at[pl.ds(s * RPS, RPS)])
        o16 = jnp.ones((HIDDEN,), jnp.float32)

        def orow(i, carry):
            ones_v[i] = o16
            return carry

        lax.fori_loop(0, CH, orow, 0)

    pltpu.sync_copy(srcw.at[wid], src_v)
    pltpu.sync_copy(dstw.at[wid], dst_v)
    plsc.subcore_barrier()

    def fire(g, rows_v, gsem):
        pltpu.async_copy(table.at[src_v.at[g]], rows_v, gsem)

    def drain(rows_v, gsem):
        pltpu.make_async_copy(table.at[src_v.at[0]], rows_v, gsem).wait()

    def scatter(g, rows_v):
        for t in range(GC):
            rows_t = rows_v.at[pl.ds(t * CH, CH)]
            pltpu.sync_copy(rows_t, acc_sh.at[dst_v.at[g * GC + t]], add=True)
            if with_count:
                pltpu.sync_copy(ones_v, cnt_sh.at[dst_v.at[g * GC + t]],
                                add=True)

    fire(0, rows0_v, gsem0)

    def pair(p, carry):
        g0 = 2 * p
        drain(rows0_v, gsem0)
        fire(g0 + 1, rows1_v, gsem1)
        scatter(g0, rows0_v)

        @pl.when(g0 + 2 < n_groups)
        def _():
            fire(g0 + 2, rows0_v, gsem0)

        drain(rows1_v, gsem1)
        scatter(g0 + 1, rows1_v)
        return carry

    lax.fori_loop(0, n_groups // 2, pair, 0)
    plsc.subcore_barrier()

    pltpu.sync_copy(acc_sh.at[pl.ds(s * RPS, RPS)], bounce_v)
    pltpu.sync_copy(bounce_v, agg_out.at[c].at[pl.ds(s * RPS, RPS)])
    if with_count:
        pltpu.sync_copy(cnt_sh.at[pl.ds(s * RPS, RPS)], bounce_v)
        pltpu.sync_copy(bounce_v, cnt_out.at[c].at[pl.ds(s * RPS, RPS)])


@functools.lru_cache(maxsize=None)
def _make_sc_pass(n_groups, with_count):
    mesh = plsc.VectorSubcoreMesh(core_axis_name="c", subcore_axis_name="s",
                                  num_cores=NC, num_subcores=NS)
    acc_t = jax.ShapeDtypeStruct((NC, NROWS, HIDDEN), jnp.float32)
    out_type = [acc_t, acc_t] if with_count else [acc_t]
    scratch = [pltpu.VMEM_SHARED((NROWS, HIDDEN), jnp.float32)]
    if with_count:
        scratch.append(pltpu.VMEM_SHARED((NROWS, HIDDEN), jnp.float32))
    scratch += [
        pltpu.VMEM((n_groups, CHG), jnp.int32),
        pltpu.VMEM((n_groups * GC, CH), jnp.int32),
        pltpu.VMEM((CHG, HIDDEN), jnp.float32),
        pltpu.VMEM((CHG, HIDDEN), jnp.float32),
    ]
    if with_count:
        scratch.append(pltpu.VMEM((CH, HIDDEN), jnp.float32))
    scratch += [
        pltpu.VMEM((RPS, HIDDEN), jnp.float32),
        pltpu.SemaphoreType.DMA,
        pltpu.SemaphoreType.DMA,
    ]
    body = functools.partial(_sc_segsum_body, n_groups, with_count)
    return pl.kernel(body, out_type=out_type, mesh=mesh,
                     scratch_types=scratch,
                     compiler_params=pltpu.CompilerParams(
                         use_tc_tiling_on_sc=False))


def _proj1_body(x_ref, wl_ref, wr_ref, p_ref, r_ref):
    x = x_ref[...]
    p_ref[...] = jnp.dot(x, wl_ref[...], preferred_element_type=jnp.float32)
    r_ref[...] = jnp.dot(x, wr_ref[...], preferred_element_type=jnp.float32)


def _h_body(agg_ref, cnt_ref, xr_ref, b1_ref, h_ref):
    a = agg_ref[0] + agg_ref[1]
    cw = cnt_ref[0] + cnt_ref[1]
    m = a[:N_NODES] / jnp.maximum(cw[:N_NODES], 1.0)
    h_ref[...] = jnp.maximum(m + xr_ref[...] + b1_ref[...], 0.0)


def _out_body(agg_ref, cnt_ref, h_ref, wl_ref, wr_ref, b2_ref, out_ref):
    a = agg_ref[0] + agg_ref[1]
    cw = cnt_ref[0] + cnt_ref[1]
    m = a[:N_NODES] / jnp.maximum(cw[:N_NODES], 1.0)
    h = h_ref[...]
    out_ref[...] = (jnp.dot(m, wl_ref[...], preferred_element_type=jnp.float32)
                    + jnp.dot(h, wr_ref[...], preferred_element_type=jnp.float32)
                    + b2_ref[...])


_proj1 = pl.pallas_call(
    _proj1_body,
    out_shape=[jax.ShapeDtypeStruct((N_NODES, HIDDEN), jnp.float32),
               jax.ShapeDtypeStruct((N_NODES, HIDDEN), jnp.float32)])

_hstep = pl.pallas_call(
    _h_body,
    out_shape=jax.ShapeDtypeStruct((N_NODES, HIDDEN), jnp.float32))

_outstep = pl.pallas_call(
    _out_body,
    out_shape=jax.ShapeDtypeStruct((N_NODES, D_OUT), jnp.float32))


def kernel(x, edge_index, W1l, b1, W1r, W2l, b2, W2r):
    n_edges = edge_index.shape[1]
    n_groups = -(-n_edges // (NW * CHG))
    n_groups += n_groups % 2
    e_pad = NW * n_groups * CHG

    src = edge_index[0].astype(jnp.int32)
    dst = edge_index[1].astype(jnp.int32)
    pad = e_pad - n_edges
    if pad:
        src = jnp.concatenate([src, jnp.zeros((pad,), jnp.int32)])
        dst = jnp.concatenate(
            [dst, N_NODES + (jnp.arange(pad, dtype=jnp.int32) % (NROWS - N_NODES))])
    srcw = src.reshape(NW, n_groups, CHG)
    dstw = dst.reshape(NW, n_groups * GC, CH)

    p1, xr = _proj1(x, W1l.T, W1r.T)
    agg1, cntw = _make_sc_pass(n_groups, True)(p1, srcw, dstw)
    h = _hstep(agg1, cntw, xr, b1.reshape(1, HIDDEN))
    agg2, = _make_sc_pass(n_groups, False)(h, srcw, dstw)
    out = _outstep(agg2, cntw, h, W2l.T, W2r.T, b2.reshape(1, D_OUT))
    return out


# speedup vs baseline: 17.4542x; 1.0740x over previous
import functools

import jax
import jax.numpy as jnp
from jax import lax
from jax.experimental import pallas as pl
from jax.experimental.pallas import tpu as pltpu
from jax.experimental.pallas import tpu_sc as plsc

N_NODES = 10000
D_FEAT = 128
HIDDEN = 16
D_OUT = 128

NC = 2
NS = 16
NW = NC * NS
CH = 128
GC = 8
CHG = CH * GC
RPS = 632
NROWS = RPS * NS


def _sc_segsum_body(n_edges, n_groups, with_count, table, ei, *rest):
    if with_count:
        (agg_out, cnt_out, acc_sh, cnt_sh, src_v, dst_v, rows0_v, rows1_v,
         ones_v, bounce_v, gsem0, gsem1) = rest
    else:
        (agg_out, acc_sh, src_v, dst_v, rows0_v, rows1_v, bounce_v,
         gsem0, gsem1) = rest
    c = lax.axis_index("c")
    s = lax.axis_index("s")
    wid = c * NS + s
    epw = n_edges // NW
    epw_pad = n_groups * CHG

    z16 = jnp.zeros((HIDDEN,), jnp.float32)

    def zrow(i, carry):
        bounce_v[i] = z16
        return carry

    lax.fori_loop(0, RPS, zrow, 0)
    pltpu.sync_copy(bounce_v, acc_sh.at[pl.ds(s * RPS, RPS)])
    if with_count:
        pltpu.sync_copy(bounce_v, cnt_sh.at[pl.ds(s * RPS, RPS)])
        o16 = jnp.ones((HIDDEN,), jnp.float32)

        def orow(i, carry):
            ones_v[i] = o16
            return carry

        lax.fori_loop(0, CH, orow, 0)

    pltpu.sync_copy(ei.at[pl.ds(wid * epw, epw)], src_v.at[pl.ds(0, epw)])
    pltpu.sync_copy(ei.at[pl.ds(n_edges + wid * epw, epw)],
                    dst_v.at[pl.ds(0, epw)])
    dummy = jnp.full((16,), N_NODES, jnp.int32) + s
    zi16 = jnp.zeros((16,), jnp.int32)
    for k in range(epw, epw_pad, 16):
        src_v[pl.ds(k, 16)] = zi16
        dst_v[pl.ds(k, 16)] = dummy
    plsc.subcore_barrier()

    def fire(g, rows_v, gsem):
        pltpu.async_copy(table.at[src_v.at[pl.ds(g * CHG, CHG)]], rows_v,
                         gsem)

    def drain(rows_v, gsem):
        pltpu.make_async_copy(table.at[src_v.at[pl.ds(0, CHG)]], rows_v,
                              gsem).wait()

    def scatter(g, rows_v):
        for t in range(GC):
            rows_t = rows_v.at[pl.ds(t * CH, CH)]
            idx = dst_v.at[pl.ds((g * GC + t) * CH, CH)]
            pltpu.sync_copy(rows_t, acc_sh.at[idx], add=True)
            if with_count:
                pltpu.sync_copy(ones_v, cnt_sh.at[idx], add=True)

    fire(0, rows0_v, gsem0)

    def pair(p, carry):
        g0 = 2 * p
        drain(rows0_v, gsem0)
        fire(g0 + 1, rows1_v, gsem1)
        scatter(g0, rows0_v)

        @pl.when(g0 + 2 < n_groups)
        def _():
            fire(g0 + 2, rows0_v, gsem0)

        drain(rows1_v, gsem1)
        scatter(g0 + 1, rows1_v)
        return carry

    lax.fori_loop(0, n_groups // 2, pair, 0)
    plsc.subcore_barrier()

    pltpu.sync_copy(acc_sh.at[pl.ds(s * RPS, RPS)], bounce_v)
    pltpu.sync_copy(bounce_v, agg_out.at[c].at[pl.ds(s * RPS, RPS)])
    if with_count:
        pltpu.sync_copy(cnt_sh.at[pl.ds(s * RPS, RPS)], bounce_v)
        pltpu.sync_copy(bounce_v, cnt_out.at[c].at[pl.ds(s * RPS, RPS)])


@functools.lru_cache(maxsize=None)
def _make_sc_pass(n_edges, n_groups, with_count):
    mesh = plsc.VectorSubcoreMesh(core_axis_name="c", subcore_axis_name="s",
                                  num_cores=NC, num_subcores=NS)
    acc_t = jax.ShapeDtypeStruct((NC, NROWS, HIDDEN), jnp.float32)
    out_type = [acc_t, acc_t] if with_count else [acc_t]
    scratch = [pltpu.VMEM_SHARED((NROWS, HIDDEN), jnp.float32)]
    if with_count:
        scratch.append(pltpu.VMEM_SHARED((NROWS, HIDDEN), jnp.float32))
    scratch += [
        pltpu.VMEM((n_groups * CHG,), jnp.int32),
        pltpu.VMEM((n_groups * CHG,), jnp.int32),
        pltpu.VMEM((CHG, HIDDEN), jnp.float32),
        pltpu.VMEM((CHG, HIDDEN), jnp.float32),
    ]
    if with_count:
        scratch.append(pltpu.VMEM((CH, HIDDEN), jnp.float32))
    scratch += [
        pltpu.VMEM((RPS, HIDDEN), jnp.float32),
        pltpu.SemaphoreType.DMA,
        pltpu.SemaphoreType.DMA,
    ]
    body = functools.partial(_sc_segsum_body, n_edges, n_groups, with_count)
    return pl.kernel(body, out_type=out_type, mesh=mesh,
                     scratch_types=scratch,
                     compiler_params=pltpu.CompilerParams(
                         use_tc_tiling_on_sc=False))


def _proj1_body(x_ref, wl_ref, wr_ref, p_ref, r_ref):
    x = x_ref[...]
    p_ref[...] = jnp.dot(x, wl_ref[...], preferred_element_type=jnp.float32)
    r_ref[...] = jnp.dot(x, wr_ref[...], preferred_element_type=jnp.float32)


def _h_body(agg_ref, cnt_ref, xr_ref, b1_ref, h_ref):
    a = agg_ref[0] + agg_ref[1]
    cw = cnt_ref[0] + cnt_ref[1]
    m = a[:N_NODES] / jnp.maximum(cw[:N_NODES], 1.0)
    h_ref[...] = jnp.maximum(m + xr_ref[...] + b1_ref[...], 0.0)


def _out_body(agg_ref, cnt_ref, h_ref, wl_ref, wr_ref, b2_ref, out_ref):
    a = agg_ref[0] + agg_ref[1]
    cw = cnt_ref[0] + cnt_ref[1]
    m = a[:N_NODES] / jnp.maximum(cw[:N_NODES], 1.0)
    h = h_ref[...]
    out_ref[...] = (jnp.dot(m, wl_ref[...], preferred_element_type=jnp.float32)
                    + jnp.dot(h, wr_ref[...], preferred_element_type=jnp.float32)
                    + b2_ref[...])


_proj1 = pl.pallas_call(
    _proj1_body,
    out_shape=[jax.ShapeDtypeStruct((N_NODES, HIDDEN), jnp.float32),
               jax.ShapeDtypeStruct((N_NODES, HIDDEN), jnp.float32)])

_hstep = pl.pallas_call(
    _h_body,
    out_shape=jax.ShapeDtypeStruct((N_NODES, HIDDEN), jnp.float32))

_outstep = pl.pallas_call(
    _out_body,
    out_shape=jax.ShapeDtypeStruct((N_NODES, D_OUT), jnp.float32))


def kernel(x, edge_index, W1l, b1, W1r, W2l, b2, W2r):
    n_edges = edge_index.shape[1]
    assert n_edges % (NW * 16) == 0
    epw = n_edges // NW
    n_groups = -(-epw // CHG)
    n_groups += n_groups % 2

    ei = edge_index.astype(jnp.int32).reshape(2 * n_edges)

    p1, xr = _proj1(x, W1l.T, W1r.T)
    agg1, cntw = _make_sc_pass(n_edges, n_groups, True)(p1, ei)
    h = _hstep(agg1, cntw, xr, b1.reshape(1, HIDDEN))
    agg2, = _make_sc_pass(n_edges, n_groups, False)(h, ei)
    out = _outstep(agg2, cntw, h, W2l.T, W2r.T, b2.reshape(1, D_OUT))
    return out


# speedup vs baseline: 17.5233x; 1.0040x over previous
import functools

import jax
import jax.numpy as jnp
from jax import lax
from jax.experimental import pallas as pl
from jax.experimental.pallas import tpu as pltpu
from jax.experimental.pallas import tpu_sc as plsc

N_NODES = 10000
D_FEAT = 128
HIDDEN = 16
D_OUT = 128

NC = 2
NS = 16
NW = NC * NS
CH = 128
GC = 8
CHG = CH * GC
RPS = 632
NROWS = RPS * NS


def _sc_segsum_body(n_edges, n_groups, with_count, table, ei, *rest):
    if with_count:
        (agg_out, cnt_out, acc_sh, cnt_sh, src_v, dst_v, rows0_v, rows1_v,
         ones_v, bounce_v, gsem0, gsem1, ssem0, ssem1) = rest
    else:
        (agg_out, acc_sh, src_v, dst_v, rows0_v, rows1_v, bounce_v,
         gsem0, gsem1, ssem0, ssem1) = rest
    c = lax.axis_index("c")
    s = lax.axis_index("s")
    wid = c * NS + s
    epw = n_edges // NW
    epw_pad = n_groups * CHG

    z16 = jnp.zeros((HIDDEN,), jnp.float32)

    def zrow(i, carry):
        bounce_v[i] = z16
        return carry

    lax.fori_loop(0, RPS, zrow, 0)
    pltpu.sync_copy(bounce_v, acc_sh.at[pl.ds(s * RPS, RPS)])
    if with_count:
        pltpu.sync_copy(bounce_v, cnt_sh.at[pl.ds(s * RPS, RPS)])
        o16 = jnp.ones((HIDDEN,), jnp.float32)

        def orow(i, carry):
            ones_v[i] = o16
            return carry

        lax.fori_loop(0, CH, orow, 0)

    pltpu.sync_copy(ei.at[pl.ds(wid * epw, epw)], src_v.at[pl.ds(0, epw)])
    pltpu.sync_copy(ei.at[pl.ds(n_edges + wid * epw, epw)],
                    dst_v.at[pl.ds(0, epw)])
    dummy = jnp.full((16,), N_NODES, jnp.int32) + s
    zi16 = jnp.zeros((16,), jnp.int32)
    for k in range(epw, epw_pad, 16):
        src_v[pl.ds(k, 16)] = zi16
        dst_v[pl.ds(k, 16)] = dummy
    plsc.subcore_barrier()

    n_sc = GC * (2 if with_count else 1)

    def fire(g, rows_v, gsem):
        pltpu.async_copy(table.at[src_v.at[pl.ds(g * CHG, CHG)]], rows_v,
                         gsem)

    def drain(rows_v, gsem):
        pltpu.make_async_copy(table.at[src_v.at[pl.ds(0, CHG)]], rows_v,
                              gsem).wait()

    def scatter(g, rows_v, ssem):
        for t in range(GC):
            rows_t = rows_v.at[pl.ds(t * CH, CH)]
            idx = dst_v.at[pl.ds((g * GC + t) * CH, CH)]
            pltpu.async_copy(rows_t, acc_sh.at[idx], ssem, add=True)
            if with_count:
                pltpu.async_copy(ones_v, cnt_sh.at[idx], ssem, add=True)

    def sdrain(rows_v, ssem):
        for _ in range(n_sc):
            pltpu.make_async_copy(rows_v.at[pl.ds(0, CH)],
                                  acc_sh.at[dst_v.at[pl.ds(0, CH)]],
                                  ssem).wait()

    fire(0, rows0_v, gsem0)

    def pair(p, carry):
        g0 = 2 * p
        drain(rows0_v, gsem0)
        fire(g0 + 1, rows1_v, gsem1)
        scatter(g0, rows0_v, ssem0)
        drain(rows1_v, gsem1)
        sdrain(rows0_v, ssem0)

        @pl.when(g0 + 2 < n_groups)
        def _():
            fire(g0 + 2, rows0_v, gsem0)

        scatter(g0 + 1, rows1_v, ssem1)
        sdrain(rows1_v, ssem1)
        return carry

    lax.fori_loop(0, n_groups // 2, pair, 0)
    plsc.subcore_barrier()

    pltpu.sync_copy(acc_sh.at[pl.ds(s * RPS, RPS)], bounce_v)
    pltpu.sync_copy(bounce_v, agg_out.at[c].at[pl.ds(s * RPS, RPS)])
    if with_count:
        pltpu.sync_copy(cnt_sh.at[pl.ds(s * RPS, RPS)], bounce_v)
        pltpu.sync_copy(bounce_v, cnt_out.at[c].at[pl.ds(s * RPS, RPS)])


@functools.lru_cache(maxsize=None)
def _make_sc_pass(n_edges, n_groups, with_count):
    mesh = plsc.VectorSubcoreMesh(core_axis_name="c", subcore_axis_name="s",
                                  num_cores=NC, num_subcores=NS)
    acc_t = jax.ShapeDtypeStruct((NC, NROWS, HIDDEN), jnp.float32)
    out_type = [acc_t, acc_t] if with_count else [acc_t]
    scratch = [pltpu.VMEM_SHARED((NROWS, HIDDEN), jnp.float32)]
    if with_count:
        scratch.append(pltpu.VMEM_SHARED((NROWS, HIDDEN), jnp.float32))
    scratch += [
        pltpu.VMEM((n_groups * CHG,), jnp.int32),
        pltpu.VMEM((n_groups * CHG,), jnp.int32),
        pltpu.VMEM((CHG, HIDDEN), jnp.float32),
        pltpu.VMEM((CHG, HIDDEN), jnp.float32),
    ]
    if with_count:
        scratch.append(pltpu.VMEM((CH, HIDDEN), jnp.float32))
    scratch += [
        pltpu.VMEM((RPS, HIDDEN), jnp.float32),
        pltpu.SemaphoreType.DMA,
        pltpu.SemaphoreType.DMA,
        pltpu.SemaphoreType.DMA,
        pltpu.SemaphoreType.DMA,
    ]
    body = functools.partial(_sc_segsum_body, n_edges, n_groups, with_count)
    return pl.kernel(body, out_type=out_type, mesh=mesh,
                     scratch_types=scratch,
                     compiler_params=pltpu.CompilerParams(
                         use_tc_tiling_on_sc=False))


def _proj1_body(x_ref, wl_ref, wr_ref, p_ref, r_ref):
    x = x_ref[...]
    p_ref[...] = jnp.dot(x, wl_ref[...], preferred_element_type=jnp.float32)
    r_ref[...] = jnp.dot(x, wr_ref[...], preferred_element_type=jnp.float32)


def _h_body(agg_ref, cnt_ref, xr_ref, b1_ref, h_ref):
    a = agg_ref[0] + agg_ref[1]
    cw = cnt_ref[0] + cnt_ref[1]
    m = a[:N_NODES] / jnp.maximum(cw[:N_NODES], 1.0)
    h_ref[...] = jnp.maximum(m + xr_ref[...] + b1_ref[...], 0.0)


def _out_body(agg_ref, cnt_ref, h_ref, wl_ref, wr_ref, b2_ref, out_ref):
    a = agg_ref[0] + agg_ref[1]
    cw = cnt_ref[0] + cnt_ref[1]
    m = a[:N_NODES] / jnp.maximum(cw[:N_NODES], 1.0)
    h = h_ref[...]
    out_ref[...] = (jnp.dot(m, wl_ref[...], preferred_element_type=jnp.float32)
                    + jnp.dot(h, wr_ref[...], preferred_element_type=jnp.float32)
                    + b2_ref[...])


_proj1 = pl.pallas_call(
    _proj1_body,
    out_shape=[jax.ShapeDtypeStruct((N_NODES, HIDDEN), jnp.float32),
               jax.ShapeDtypeStruct((N_NODES, HIDDEN), jnp.float32)])

_hstep = pl.pallas_call(
    _h_body,
    out_shape=jax.ShapeDtypeStruct((N_NODES, HIDDEN), jnp.float32))

_outstep = pl.pallas_call(
    _out_body,
    out_shape=jax.ShapeDtypeStruct((N_NODES, D_OUT), jnp.float32))


def kernel(x, edge_index, W1l, b1, W1r, W2l, b2, W2r):
    n_edges = edge_index.shape[1]
    assert n_edges % (NW * 16) == 0
    epw = n_edges // NW
    n_groups = -(-epw // CHG)
    n_groups += n_groups % 2

    ei = edge_index.astype(jnp.int32).reshape(2 * n_edges)

    p1, xr = _proj1(x, W1l.T, W1r.T)
    agg1, cntw = _make_sc_pass(n_edges, n_groups, True)(p1, ei)
    h = _hstep(agg1, cntw, xr, b1.reshape(1, HIDDEN))
    agg2, = _make_sc_pass(n_edges, n_groups, False)(h, ei)
    out = _outstep(agg2, cntw, h, W2l.T, W2r.T, b2.reshape(1, D_OUT))
    return out


# speedup vs baseline: 18.2840x; 1.0434x over previous
import functools

import jax
import jax.numpy as jnp
from jax import lax
from jax.experimental import pallas as pl
from jax.experimental.pallas import tpu as pltpu
from jax.experimental.pallas import tpu_sc as plsc

N_NODES = 10000
D_FEAT = 128
HIDDEN = 16
D_OUT = 128

NC = 2
NS = 16
NW = NC * NS
CH = 128
GC = 8
CHG = CH * GC
NBF = 4
RPS = 632
NROWS = RPS * NS


def _sc_segsum_body(n_edges, n_groups, with_count, table, ei, *rest):
    if with_count:
        (agg_out, cnt_out, acc_sh, cnt_sh, src_v, dst_v, rows_v,
         ones_v, bounce_v, gsems, ssems) = rest
    else:
        (agg_out, acc_sh, src_v, dst_v, rows_v, bounce_v,
         gsems, ssems) = rest
    c = lax.axis_index("c")
    s = lax.axis_index("s")
    wid = c * NS + s
    epw = n_edges // NW
    epw_pad = n_groups * CHG

    z16 = jnp.zeros((HIDDEN,), jnp.float32)

    def zrow(i, carry):
        bounce_v[i] = z16
        return carry

    lax.fori_loop(0, RPS, zrow, 0)
    pltpu.sync_copy(bounce_v, acc_sh.at[pl.ds(s * RPS, RPS)])
    if with_count:
        pltpu.sync_copy(bounce_v, cnt_sh.at[pl.ds(s * RPS, RPS)])
        o16 = jnp.ones((HIDDEN,), jnp.float32)

        def orow(i, carry):
            ones_v[i] = o16
            return carry

        lax.fori_loop(0, CH, orow, 0)

    pltpu.sync_copy(ei.at[pl.ds(wid * epw, epw)], src_v.at[pl.ds(0, epw)])
    pltpu.sync_copy(ei.at[pl.ds(n_edges + wid * epw, epw)],
                    dst_v.at[pl.ds(0, epw)])
    dummy = jnp.full((16,), N_NODES, jnp.int32) + s
    zi16 = jnp.zeros((16,), jnp.int32)
    for k in range(epw, epw_pad, 16):
        src_v[pl.ds(k, 16)] = zi16
        dst_v[pl.ds(k, 16)] = dummy
    plsc.subcore_barrier()

    n_sc = GC * (2 if with_count else 1)

    def fire(g, b):
        pltpu.async_copy(table.at[src_v.at[pl.ds(g * CHG, CHG)]],
                         rows_v.at[b], gsems.at[b])

    def gdrain(b):
        pltpu.make_async_copy(table.at[src_v.at[pl.ds(0, CHG)]],
                              rows_v.at[b], gsems.at[b]).wait()

    def scatter(g, b):
        for t in range(GC):
            rows_t = rows_v.at[b].at[pl.ds(t * CH, CH)]
            idx = dst_v.at[pl.ds((g * GC + t) * CH, CH)]
            pltpu.async_copy(rows_t, acc_sh.at[idx], ssems.at[b], add=True)
            if with_count:
                pltpu.async_copy(ones_v, cnt_sh.at[idx], ssems.at[b],
                                 add=True)

    def sdrain(b):
        for _ in range(n_sc):
            pltpu.make_async_copy(rows_v.at[b].at[pl.ds(0, CH)],
                                  acc_sh.at[dst_v.at[pl.ds(0, CH)]],
                                  ssems.at[b]).wait()

    for g in range(min(NBF - 1, n_groups)):
        fire(g, g)

    def step(g, carry):
        b = g % NBF
        gdrain(b)
        scatter(g, b)

        @pl.when(g + NBF - 1 < n_groups)
        def _():
            bn = (g + NBF - 1) % NBF

            @pl.when(g >= 1)
            def _():
                sdrain(bn)

            fire(g + NBF - 1, bn)

        return carry

    lax.fori_loop(0, n_groups, step, 0)
    for k in range(min(NBF, n_groups)):
        sdrain((n_groups - 1 - k) % NBF)
    plsc.subcore_barrier()

    pltpu.sync_copy(acc_sh.at[pl.ds(s * RPS, RPS)], bounce_v)
    pltpu.sync_copy(bounce_v, agg_out.at[c].at[pl.ds(s * RPS, RPS)])
    if with_count:
        pltpu.sync_copy(cnt_sh.at[pl.ds(s * RPS, RPS)], bounce_v)
        pltpu.sync_copy(bounce_v, cnt_out.at[c].at[pl.ds(s * RPS, RPS)])


@functools.lru_cache(maxsize=None)
def _make_sc_pass(n_edges, n_groups, with_count):
    mesh = plsc.VectorSubcoreMesh(core_axis_name="c", subcore_axis_name="s",
                                  num_cores=NC, num_subcores=NS)
    acc_t = jax.ShapeDtypeStruct((NC, NROWS, HIDDEN), jnp.float32)
    out_type = [acc_t, acc_t] if with_count else [acc_t]
    scratch = [pltpu.VMEM_SHARED((NROWS, HIDDEN), jnp.float32)]
    if with_count:
        scratch.append(pltpu.VMEM_SHARED((NROWS, HIDDEN), jnp.float32))
    scratch += [
        pltpu.VMEM((n_groups * CHG,), jnp.int32),
        pltpu.VMEM((n_groups * CHG,), jnp.int32),
        pltpu.VMEM((NBF, CHG, HIDDEN), jnp.float32),
    ]
    if with_count:
        scratch.append(pltpu.VMEM((CH, HIDDEN), jnp.float32))
    scratch += [
        pltpu.VMEM((RPS, HIDDEN), jnp.float32),
        pltpu.SemaphoreType.DMA((NBF,)),
        pltpu.SemaphoreType.DMA((NBF,)),
    ]
    body = functools.partial(_sc_segsum_body, n_edges, n_groups, with_count)
    return pl.kernel(body, out_type=out_type, mesh=mesh,
                     scratch_types=scratch,
                     compiler_params=pltpu.CompilerParams(
                         use_tc_tiling_on_sc=False))


def _proj1_body(x_ref, wl_ref, wr_ref, p_ref, r_ref):
    x = x_ref[...]
    p_ref[...] = jnp.dot(x, wl_ref[...], preferred_element_type=jnp.float32)
    r_ref[...] = jnp.dot(x, wr_ref[...], preferred_element_type=jnp.float32)


def _h_body(agg_ref, cnt_ref, xr_ref, b1_ref, h_ref):
    a = agg_ref[0] + agg_ref[1]
    cw = cnt_ref[0] + cnt_ref[1]
    m = a[:N_NODES] / jnp.maximum(cw[:N_NODES], 1.0)
    h_ref[...] = jnp.maximum(m + xr_ref[...] + b1_ref[...], 0.0)


def _out_body(agg_ref, cnt_ref, h_ref, wl_ref, wr_ref, b2_ref, out_ref):
    a = agg_ref[0] + agg_ref[1]
    cw = cnt_ref[0] + cnt_ref[1]
    m = a[:N_NODES] / jnp.maximum(cw[:N_NODES], 1.0)
    h = h_ref[...]
    out_ref[...] = (jnp.dot(m, wl_ref[...], preferred_element_type=jnp.float32)
                    + jnp.dot(h, wr_ref[...], preferred_element_type=jnp.float32)
                    + b2_ref[...])


_proj1 = pl.pallas_call(
    _proj1_body,
    out_shape=[jax.ShapeDtypeStruct((N_NODES, HIDDEN), jnp.float32),
               jax.ShapeDtypeStruct((N_NODES, HIDDEN), jnp.float32)])

_hstep = pl.pallas_call(
    _h_body,
    out_shape=jax.ShapeDtypeStruct((N_NODES, HIDDEN), jnp.float32))

_outstep = pl.pallas_call(
    _out_body,
    out_shape=jax.ShapeDtypeStruct((N_NODES, D_OUT), jnp.float32))


def kernel(x, edge_index, W1l, b1, W1r, W2l, b2, W2r):
    n_edges = edge_index.shape[1]
    assert n_edges % (NW * 16) == 0
    epw = n_edges // NW
    n_groups = -(-epw // CHG)
    n_groups += n_groups % 2

    ei = edge_index.astype(jnp.int32).reshape(2 * n_edges)

    p1, xr = _proj1(x, W1l.T, W1r.T)
    agg1, cntw = _make_sc_pass(n_edges, n_groups, True)(p1, ei)
    h = _hstep(agg1, cntw, xr, b1.reshape(1, HIDDEN))
    agg2, = _make_sc_pass(n_edges, n_groups, False)(h, ei)
    out = _outstep(agg2, cntw, h, W2l.T, W2r.T, b2.reshape(1, D_OUT))
    return out


# speedup vs baseline: 23.3099x; 1.2749x over previous
import functools

import jax
import jax.numpy as jnp
from jax import lax
from jax.experimental import pallas as pl
from jax.experimental.pallas import tpu as pltpu
from jax.experimental.pallas import tpu_sc as plsc

N_NODES = 10000
D_FEAT = 128
HIDDEN = 16
D_OUT = 128

NC = 2
NS = 16
NW = NC * NS
CH = 128
GC = 8
CHG = CH * GC
NBF = 4
RPS = 632
NROWS = RPS * NS


def _sc_segsum_body(n_edges, n_groups, with_count, table, ei, *rest):
    if with_count:
        (agg_out, cnt_out, acc_sh, cnt_sh, src_v, dst_v, rows_v,
         ones_v, bounce_v, gsems, ssems) = rest
    else:
        (agg_out, acc_sh, src_v, dst_v, rows_v, bounce_v,
         gsems, ssems) = rest
    c = lax.axis_index("c")
    s = lax.axis_index("s")
    wid = c * NS + s
    epw = n_edges // NW
    epw_pad = n_groups * CHG

    z16 = jnp.zeros((HIDDEN,), jnp.float32)

    def zrow(i, carry):
        bounce_v[i] = z16
        return carry

    lax.fori_loop(0, RPS, zrow, 0)
    pltpu.sync_copy(bounce_v, acc_sh.at[pl.ds(s * RPS, RPS)])
    if with_count:
        pltpu.sync_copy(bounce_v, cnt_sh.at[pl.ds(s * RPS, RPS)])
        o16 = jnp.ones((HIDDEN,), jnp.float32)

        def orow(i, carry):
            ones_v[i] = o16
            return carry

        lax.fori_loop(0, CH, orow, 0)

    pltpu.sync_copy(ei.at[pl.ds(wid * epw, epw)], src_v.at[pl.ds(0, epw)])
    pltpu.sync_copy(ei.at[pl.ds(n_edges + wid * epw, epw)],
                    dst_v.at[pl.ds(0, epw)])
    dummy = jnp.full((16,), N_NODES, jnp.int32) + s
    zi16 = jnp.zeros((16,), jnp.int32)
    for k in range(epw, epw_pad, 16):
        src_v[pl.ds(k, 16)] = zi16
        dst_v[pl.ds(k, 16)] = dummy
    plsc.subcore_barrier()

    n_sc = GC * (2 if with_count else 1)

    def fire(g, b):
        pltpu.async_copy(table.at[src_v.at[pl.ds(g * CHG, CHG)]],
                         rows_v.at[b], gsems.at[b])

    def gdrain(b):
        pltpu.make_async_copy(table.at[src_v.at[pl.ds(0, CHG)]],
                              rows_v.at[b], gsems.at[b]).wait()

    def scatter(g, b):
        for t in range(GC):
            rows_t = rows_v.at[b].at[pl.ds(t * CH, CH)]
            idx = dst_v.at[pl.ds((g * GC + t) * CH, CH)]
            pltpu.async_copy(rows_t, acc_sh.at[idx], ssems.at[b], add=True)
            if with_count:
                pltpu.async_copy(ones_v, cnt_sh.at[idx], ssems.at[b],
                                 add=True)

    def sdrain(b):
        for _ in range(n_sc):
            pltpu.make_async_copy(rows_v.at[b].at[pl.ds(0, CH)],
                                  acc_sh.at[dst_v.at[pl.ds(0, CH)]],
                                  ssems.at[b]).wait()

    for g in range(min(NBF - 1, n_groups)):
        fire(g, g)

    def step(g, carry):
        b = g % NBF
        gdrain(b)
        scatter(g, b)

        @pl.when(g + NBF - 1 < n_groups)
        def _():
            bn = (g + NBF - 1) % NBF

            @pl.when(g >= 1)
            def _():
                sdrain(bn)

            fire(g + NBF - 1, bn)

        return carry

    lax.fori_loop(0, n_groups, step, 0)
    for k in range(min(NBF, n_groups)):
        sdrain((n_groups - 1 - k) % NBF)
    plsc.subcore_barrier()

    pltpu.sync_copy(acc_sh.at[pl.ds(s * RPS, RPS)], bounce_v)
    pltpu.sync_copy(bounce_v, agg_out.at[c].at[pl.ds(s * RPS, RPS)])
    if with_count:
        pltpu.sync_copy(cnt_sh.at[pl.ds(s * RPS, RPS)], bounce_v)
        pltpu.sync_copy(bounce_v, cnt_out.at[c].at[pl.ds(s * RPS, RPS)])


@functools.lru_cache(maxsize=None)
def _make_sc_pass(n_edges, n_groups, with_count):
    mesh = plsc.VectorSubcoreMesh(core_axis_name="c", subcore_axis_name="s",
                                  num_cores=NC, num_subcores=NS)
    acc_t = jax.ShapeDtypeStruct((NC, NROWS, HIDDEN), jnp.float32)
    out_type = [acc_t, acc_t] if with_count else [acc_t]
    scratch = [pltpu.VMEM_SHARED((NROWS, HIDDEN), jnp.float32)]
    if with_count:
        scratch.append(pltpu.VMEM_SHARED((NROWS, HIDDEN), jnp.float32))
    scratch += [
        pltpu.VMEM((n_groups * CHG,), jnp.int32),
        pltpu.VMEM((n_groups * CHG,), jnp.int32),
        pltpu.VMEM((NBF, CHG, HIDDEN), jnp.float32),
    ]
    if with_count:
        scratch.append(pltpu.VMEM((CH, HIDDEN), jnp.float32))
    scratch += [
        pltpu.VMEM((RPS, HIDDEN), jnp.float32),
        pltpu.SemaphoreType.DMA((NBF,)),
        pltpu.SemaphoreType.DMA((NBF,)),
    ]
    body = functools.partial(_sc_segsum_body, n_edges, n_groups, with_count)
    return pl.kernel(body, out_type=out_type, mesh=mesh,
                     scratch_types=scratch,
                     compiler_params=pltpu.CompilerParams(
                         use_tc_tiling_on_sc=False))


NPK = N_NODES // 8
APK = NROWS // 8


def _proj1_body(x_ref, vl_ref, vr_ref, p_ref, r_ref):
    xp = jnp.reshape(x_ref[...], (NPK, 8 * D_FEAT))
    p_ref[...] = jnp.dot(xp, vl_ref[...], preferred_element_type=jnp.float32)
    r_ref[...] = jnp.dot(xp, vr_ref[...], preferred_element_type=jnp.float32)


def _h_body(agg_ref, cnt_ref, xr_ref, b1_ref, h_ref):
    a = agg_ref[0] + agg_ref[1]
    cw = cnt_ref[0] + cnt_ref[1]
    m = a[:NPK] / jnp.maximum(cw[:NPK], 1.0)
    h_ref[...] = jnp.maximum(m + xr_ref[...] + b1_ref[...], 0.0)


def _out_body(agg_ref, cnt_ref, h_ref, vl_ref, vr_ref, b2_ref, out_ref):
    a = agg_ref[0] + agg_ref[1]
    cw = cnt_ref[0] + cnt_ref[1]
    m = a[:NPK] / jnp.maximum(cw[:NPK], 1.0)
    outp = (jnp.dot(m, vl_ref[...], preferred_element_type=jnp.float32)
            + jnp.dot(h_ref[...], vr_ref[...], preferred_element_type=jnp.float32)
            + b2_ref[...])
    out_ref[...] = jnp.reshape(outp, (N_NODES, D_OUT))


_proj1 = pl.pallas_call(
    _proj1_body,
    out_shape=[jax.ShapeDtypeStruct((NPK, 8 * HIDDEN), jnp.float32),
               jax.ShapeDtypeStruct((NPK, 8 * HIDDEN), jnp.float32)])

_hstep = pl.pallas_call(
    _h_body,
    out_shape=jax.ShapeDtypeStruct((NPK, 8 * HIDDEN), jnp.float32))

_outstep = pl.pallas_call(
    _out_body,
    out_shape=jax.ShapeDtypeStruct((N_NODES, D_OUT), jnp.float32))


def _blockdiag(w, reps=8):
    return jax.scipy.linalg.block_diag(*([w] * reps))


def kernel(x, edge_index, W1l, b1, W1r, W2l, b2, W2r):
    n_edges = edge_index.shape[1]
    assert n_edges % (NW * 16) == 0
    epw = n_edges // NW
    n_groups = -(-epw // CHG)
    n_groups += n_groups % 2

    ei = edge_index.astype(jnp.int32).reshape(2 * n_edges)

    v1l = _blockdiag(W1l.T)
    v1r = _blockdiag(W1r.T)
    v2l = _blockdiag(W2l.T)
    v2r = _blockdiag(W2r.T)
    b1t = jnp.tile(b1, 8).reshape(1, 8 * HIDDEN)
    b2t = jnp.tile(b2, 8).reshape(1, 8 * D_OUT)

    p1p, xrp = _proj1(x, v1l, v1r)
    sc1 = _make_sc_pass(n_edges, n_groups, True)
    agg1, cntw = sc1(p1p.reshape(N_NODES, HIDDEN), ei)
    agg1p = agg1.reshape(NC, APK, 8 * HIDDEN)
    cntp = cntw.reshape(NC, APK, 8 * HIDDEN)
    hp = _hstep(agg1p, cntp, xrp, b1t)
    sc2 = _make_sc_pass(n_edges, n_groups, False)
    agg2, = sc2(hp.reshape(N_NODES, HIDDEN), ei)
    agg2p = agg2.reshape(NC, APK, 8 * HIDDEN)
    out = _outstep(agg2p, cntp, hp, v2l, v2r, b2t)
    return out
